# baseline (speedup 1.0000x reference)
import functools

import jax
import jax.numpy as jnp
from jax import lax
from jax.experimental import pallas as pl
from jax.experimental.pallas import tpu as pltpu
from jax.experimental.pallas import tpu_sc as plsc

VOCAB = 1000000
EMBED_DIM = 300
D_PAD = 384
D_OUT = 304
B, NSENT, SLEN = 1024, 10, 20
NS_TOT = NSENT * SLEN
TOTAL = B * NS_TOT

_info = plsc.get_sparse_core_info()
_NC, _NS = _info.num_cores, _info.num_subcores
_NW = _NC * _NS

_VBLK = 8192
_NVBLK = -(-VOCAB // _VBLK)


def _table_transpose_body(tab_t_ref, tab_r_ref):
    x = tab_t_ref[...]
    xt = jnp.swapaxes(x, 0, 1)
    tab_r_ref[...] = jnp.pad(xt, ((0, 0), (0, D_PAD - EMBED_DIM)))


_table_transpose = pl.pallas_call(
    _table_transpose_body,
    out_shape=jax.ShapeDtypeStruct((VOCAB, D_PAD), jnp.float32),
    grid=(_NVBLK,),
    in_specs=[pl.BlockSpec((EMBED_DIM, _VBLK), lambda i: (0, i))],
    out_specs=pl.BlockSpec((_VBLK, D_PAD), lambda i: (i, 0)),
)

TOTAL2 = 2 * TOTAL
_PER_W = TOTAL2 // _NW
_CHUNK = 128
_NCHUNK = _PER_W // _CHUNK


@functools.partial(
    pl.kernel,
    out_type=jax.ShapeDtypeStruct((TOTAL2, D_PAD), jnp.float32),
    mesh=plsc.VectorSubcoreMesh(core_axis_name="c", subcore_axis_name="s"),
    scratch_types=[
        pltpu.VMEM((_PER_W,), jnp.int32),
        pltpu.VMEM((_CHUNK, D_PAD), jnp.float32),
        pltpu.VMEM((_CHUNK, D_PAD), jnp.float32),
        pltpu.SemaphoreType.DMA,
        pltpu.SemaphoreType.DMA,
    ],
)
def _embed_gather(table_hbm, idx_hbm, out_hbm, idx_v, rows0, rows1,
                  sem0, sem1):
    wid = lax.axis_index("s") * _NC + lax.axis_index("c")
    base = wid * _PER_W
    pltpu.sync_copy(idx_hbm.at[pl.ds(base, _PER_W)], idx_v)

    def _idx(i):
        off = pl.multiple_of(i * _CHUNK, _CHUNK)
        return idx_v.at[pl.ds(off, _CHUNK)]

    def start(i, rows, sem):
        pltpu.async_copy(table_hbm.at[_idx(i)], rows, sem)

    def finish(i, rows, sem):
        pltpu.make_async_copy(table_hbm.at[_idx(i)], rows, sem).wait()
        off = pl.multiple_of(base + i * _CHUNK, _CHUNK)
        pltpu.sync_copy(rows, out_hbm.at[pl.ds(off, _CHUNK)])

    start(0, rows0, sem0)

    def body(j, carry):
        i0 = 2 * j
        start(i0 + 1, rows1, sem1)
        finish(i0, rows0, sem0)

        @pl.when(j < _NCHUNK // 2 - 1)
        def _():
            start(i0 + 2, rows0, sem0)

        finish(i0 + 1, rows1, sem1)
        return carry

    lax.fori_loop(0, _NCHUNK // 2, body, 0)


def _out_transpose_body(rows_ref, out_ref):
    x = rows_ref[...]
    xt = jnp.swapaxes(x, 0, 1)
    out_ref[...] = xt[:D_OUT, :]


_out_transpose = pl.pallas_call(
    _out_transpose_body,
    out_shape=jax.ShapeDtypeStruct((2 * NS_TOT * D_OUT, B), jnp.float32),
    grid=(2 * NS_TOT,),
    in_specs=[pl.BlockSpec((B, D_PAD), lambda i: (i, 0))],
    out_specs=pl.BlockSpec((D_OUT, B), lambda i: (i, 0)),
)


def _finish_side(out_t):
    out4 = out_t.reshape(NSENT, SLEN, D_OUT, B)[:, :, :EMBED_DIM, :]
    return out4.transpose(3, 0, 1, 2)


def kernel(X_left, X_right, embed_weight):
    tab_t = embed_weight.T
    tab_r = _table_transpose(tab_t)
    idx_l = X_left.transpose(1, 2, 0).reshape(-1)
    idx_r = X_right.transpose(1, 2, 0).reshape(-1)
    idx = jnp.concatenate([idx_l, idx_r])
    rows = _embed_gather(tab_r, idx)
    out_t = _out_transpose(rows)
    half = NS_TOT * D_OUT
    return (_finish_side(out_t[:half]), _finish_side(out_t[half:]))


# speedup vs baseline: 1.1827x; 1.1827x over previous
import functools

import jax
import jax.numpy as jnp
from jax import lax
from jax.experimental import pallas as pl
from jax.experimental.pallas import tpu as pltpu
from jax.experimental.pallas import tpu_sc as plsc

VOCAB = 1000000
EMBED_DIM = 300
D_PAD = 384
D_OUT = 304
B, NSENT, SLEN = 1024, 10, 20
NS_TOT = NSENT * SLEN
TOTAL = B * NS_TOT

_info = plsc.get_sparse_core_info()
_NC, _NS = _info.num_cores, _info.num_subcores
_NW = _NC * _NS

_VBLK = 8192
_NVBLK = -(-VOCAB // _VBLK)


def _table_transpose_body(tab_t_ref, tab_r_ref):
    x = tab_t_ref[...]
    xt = jnp.swapaxes(x, 0, 1)
    tab_r_ref[...] = jnp.pad(xt, ((0, 0), (0, D_PAD - EMBED_DIM)))


_table_transpose = pl.pallas_call(
    _table_transpose_body,
    out_shape=jax.ShapeDtypeStruct((VOCAB, D_PAD), jnp.float32),
    grid=(_NVBLK,),
    in_specs=[pl.BlockSpec((EMBED_DIM, _VBLK), lambda i: (0, i))],
    out_specs=pl.BlockSpec((_VBLK, D_PAD), lambda i: (i, 0)),
)

_PER_W = TOTAL // _NW
_CHUNK = 128
_NCHUNK = _PER_W // _CHUNK


@functools.partial(
    pl.kernel,
    out_type=jax.ShapeDtypeStruct((TOTAL, D_PAD), jnp.float32),
    mesh=plsc.VectorSubcoreMesh(core_axis_name="c", subcore_axis_name="s"),
    scratch_types=[
        pltpu.VMEM((_PER_W,), jnp.int32),
        pltpu.VMEM((_CHUNK, D_PAD), jnp.float32),
        pltpu.VMEM((_CHUNK, D_PAD), jnp.float32),
        pltpu.SemaphoreType.DMA,
        pltpu.SemaphoreType.DMA,
    ],
)
def _embed_gather(table_hbm, idx_hbm, out_hbm, idx_v, rows0, rows1,
                  sem0, sem1):
    wid = lax.axis_index("s") * _NC + lax.axis_index("c")
    base = wid * _PER_W
    pltpu.sync_copy(idx_hbm.at[pl.ds(base, _PER_W)], idx_v)

    def _idx(i):
        off = pl.multiple_of(i * _CHUNK, _CHUNK)
        return idx_v.at[pl.ds(off, _CHUNK)]

    def start(i, rows, sem):
        pltpu.async_copy(table_hbm.at[_idx(i)], rows, sem)

    def finish(i, rows, sem):
        pltpu.make_async_copy(table_hbm.at[_idx(i)], rows, sem).wait()
        off = pl.multiple_of(base + i * _CHUNK, _CHUNK)
        pltpu.sync_copy(rows, out_hbm.at[pl.ds(off, _CHUNK)])

    start(0, rows0, sem0)

    def body(j, carry):
        i0 = 2 * j
        start(i0 + 1, rows1, sem1)
        finish(i0, rows0, sem0)

        @pl.when(j < _NCHUNK // 2 - 1)
        def _():
            start(i0 + 2, rows0, sem0)

        finish(i0 + 1, rows1, sem1)
        return carry

    lax.fori_loop(0, _NCHUNK // 2, body, 0)


def _out_transpose_body(rows_ref, out_ref):
    x = rows_ref[...]
    xt = jnp.swapaxes(x, 0, 1)
    out_ref[...] = xt[:D_OUT, :]


_out_transpose = pl.pallas_call(
    _out_transpose_body,
    out_shape=jax.ShapeDtypeStruct((NS_TOT * D_OUT, B), jnp.float32),
    grid=(NS_TOT,),
    in_specs=[pl.BlockSpec((B, D_PAD), lambda i: (i, 0))],
    out_specs=pl.BlockSpec((D_OUT, B), lambda i: (i, 0)),
)


def _finish_side(rows):
    out_t = _out_transpose(rows)
    out4 = out_t.reshape(NSENT, SLEN, D_OUT, B)[:, :, :EMBED_DIM, :]
    return out4.transpose(3, 0, 1, 2)


def kernel(X_left, X_right, embed_weight):
    tab_t = embed_weight.T
    tab_r = _table_transpose(tab_t)
    idx_l = X_left.transpose(1, 2, 0).reshape(-1)
    idx_r = X_right.transpose(1, 2, 0).reshape(-1)
    rows_l = _embed_gather(tab_r, idx_l)
    rows_r = _embed_gather(tab_r, idx_r)
    return (_finish_side(rows_l), _finish_side(rows_r))
